# Initial kernel scaffold; baseline (speedup 1.0000x reference)
import jax
import jax.numpy as jnp
from jax.experimental import pallas as pl


def kernel(queries, keys, k):
    raise NotImplementedError("write your pallas kernel here")


# speedup vs baseline: 2.4576x; 2.4576x over previous
import functools

import jax
import jax.numpy as jnp
from jax.experimental import pallas as pl
from jax.experimental.pallas import tpu as pltpu

BK = 2048
TOPK = 10
PADW = 16
NEG = float("-inf")


def _retrieval_kernel(q_ref, k_ref,
                      ms_ref, ti_ref, rr_ref, mi_ref, ai_ref, ts_ref,
                      s_ref, raw_ref, qn_ref, rc_ref, ri_ref, rk_ref,
                      *, K):
    j = pl.program_id(0)
    nb = pl.num_programs(0)
    Q = q_ref.shape[0]

    @pl.when(j == 0)
    def _init():
        q = q_ref[...]
        qnorm = jnp.sqrt(jnp.sum(q * q, axis=1, keepdims=True))
        qn_ref[...] = q / (qnorm + 1e-12)
        rc_ref[...] = jnp.full((Q, PADW), NEG, jnp.float32)
        ri_ref[...] = jnp.zeros((Q, PADW), jnp.int32)
        rk_ref[...] = jnp.zeros((Q, PADW), jnp.float32)

    kb = k_ref[...]
    knorm = jnp.sqrt(jnp.sum(kb * kb, axis=1, keepdims=True))
    knb = kb / (knorm + 1e-12)
    cos = jax.lax.dot_general(qn_ref[...].astype(jnp.bfloat16),
                              knb.astype(jnp.bfloat16),
                              (((1,), (1,)), ((), ())),
                              preferred_element_type=jnp.float32)
    raw_ref[...] = jax.lax.dot_general(q_ref[...].astype(jnp.bfloat16),
                                       kb.astype(jnp.bfloat16),
                                       (((1,), (1,)), ((), ())),
                                       preferred_element_type=jnp.float32)

    colid = jax.lax.broadcasted_iota(jnp.int32, (Q, BK), 1)
    valid = (colid + j * BK) < K
    thresh = rc_ref[:, TOPK - 1:TOPK]
    s = jnp.where(valid & (cos > thresh), cos, NEG)
    s_ref[...] = s
    cnt = jnp.sum((s > NEG).astype(jnp.int32), axis=1, keepdims=True)
    trips = jnp.minimum(jnp.max(cnt), TOPK)

    lanep = jax.lax.broadcasted_iota(jnp.int32, (Q, PADW), 1)

    def ext_body(_, carry):
        sb = s_ref[...]
        m = jnp.max(sb, axis=1, keepdims=True)
        cc = jnp.where(sb == m, colid, jnp.int32(2**30))
        col = jnp.min(cc, axis=1, keepdims=True)
        hit = colid == col
        s_ref[...] = jnp.where(hit, NEG, sb)
        rawsel = jnp.max(jnp.where(hit, raw_ref[...], NEG), axis=1,
                         keepdims=True)
        gidx = col + j * BK
        rv = rc_ref[...]
        ri = ri_ref[...]
        rk = rk_ref[...]
        ge = (rv >= m) & (lanep < TOPK)
        pos = jnp.sum(ge.astype(jnp.int32), axis=1, keepdims=True)
        rv_s = jnp.concatenate([jnp.full((Q, 1), NEG, jnp.float32),
                                rv[:, :PADW - 1]], axis=1)
        ri_s = jnp.concatenate([jnp.zeros((Q, 1), jnp.int32),
                                ri[:, :PADW - 1]], axis=1)
        rk_s = jnp.concatenate([jnp.zeros((Q, 1), jnp.float32),
                                rk[:, :PADW - 1]], axis=1)
        at = lanep == pos
        lt = lanep < pos
        rc_ref[...] = jnp.where(lt, rv, jnp.where(at, m, rv_s))
        ri_ref[...] = jnp.where(lt, ri, jnp.where(at, gidx, ri_s))
        rk_ref[...] = jnp.where(lt, rk, jnp.where(at, rawsel, rk_s))
        return carry

    jax.lax.fori_loop(0, trips, ext_body, 0)

    @pl.when(j == nb - 1)
    def _fin():
        rv = rc_ref[:, :TOPK]
        ri = ri_ref[:, :TOPK]
        rr = rk_ref[:, :TOPK]
        mx = jnp.max(rr, axis=1, keepdims=True)
        lane10 = jax.lax.broadcasted_iota(jnp.int32, (Q, TOPK), 1)
        cc = jnp.where(rr == mx, lane10, jnp.int32(99))
        mi = jnp.min(cc, axis=1, keepdims=True)
        ans = jnp.sum(jnp.where(lane10 == mi, ri, 0), axis=1, keepdims=True)
        ans = jnp.where(mx > 0.5, ans, -1)
        ms_ref[...] = mx
        ti_ref[...] = ri
        rr_ref[...] = rr
        mi_ref[...] = mi
        ai_ref[...] = ans
        ts_ref[...] = rv


def kernel(queries, keys, k):
    Q, D = queries.shape
    K = keys.shape[0]
    nb = pl.cdiv(K, BK)
    out_shape = [
        jax.ShapeDtypeStruct((Q, 1), jnp.float32),
        jax.ShapeDtypeStruct((Q, TOPK), jnp.int32),
        jax.ShapeDtypeStruct((Q, TOPK), jnp.float32),
        jax.ShapeDtypeStruct((Q, 1), jnp.int32),
        jax.ShapeDtypeStruct((Q, 1), jnp.int32),
        jax.ShapeDtypeStruct((Q, TOPK), jnp.float32),
    ]
    outs = pl.pallas_call(
        functools.partial(_retrieval_kernel, K=K),
        grid=(nb,),
        in_specs=[
            pl.BlockSpec((Q, D), lambda j: (0, 0)),
            pl.BlockSpec((BK, D), lambda j: (j, 0)),
        ],
        out_specs=[pl.BlockSpec(s.shape, lambda j: (0, 0)) for s in out_shape],
        out_shape=out_shape,
        scratch_shapes=[
            pltpu.VMEM((Q, BK), jnp.float32),
            pltpu.VMEM((Q, BK), jnp.float32),
            pltpu.VMEM((Q, D), jnp.float32),
            pltpu.VMEM((Q, PADW), jnp.float32),
            pltpu.VMEM((Q, PADW), jnp.int32),
            pltpu.VMEM((Q, PADW), jnp.float32),
        ],
    )(queries, keys)
    ms, ti, rr, mi, ai, ts = outs
    return (ms[:, 0], ti, rr, mi[:, 0], ai[:, 0], ts)


# speedup vs baseline: 2.6546x; 1.0802x over previous
import functools

import jax
import jax.numpy as jnp
from jax.experimental import pallas as pl
from jax.experimental.pallas import tpu as pltpu

BK = 1024
TOPK = 10
PADW = 16
NEG = float("-inf")


def _retrieval_kernel(q_ref, k_ref,
                      ms_ref, ti_ref, rr_ref, mi_ref, ai_ref, ts_ref,
                      s_ref, raw_ref, qn_ref, rc_ref, ri_ref, rk_ref,
                      *, K):
    j = pl.program_id(0)
    nb = pl.num_programs(0)
    Q = q_ref.shape[0]

    @pl.when(j == 0)
    def _init():
        q = q_ref[...]
        qnorm = jnp.sqrt(jnp.sum(q * q, axis=1, keepdims=True))
        qn_ref[...] = q / (qnorm + 1e-12)
        rc_ref[...] = jnp.full((Q, PADW), NEG, jnp.float32)
        ri_ref[...] = jnp.zeros((Q, PADW), jnp.int32)
        rk_ref[...] = jnp.zeros((Q, PADW), jnp.float32)

    kb = k_ref[...]
    knorm = jnp.sqrt(jnp.sum(kb * kb, axis=1, keepdims=True))
    knb = kb / (knorm + 1e-12)
    cos = jax.lax.dot_general(qn_ref[...].astype(jnp.bfloat16),
                              knb.astype(jnp.bfloat16),
                              (((1,), (1,)), ((), ())),
                              preferred_element_type=jnp.float32)
    raw_ref[...] = jax.lax.dot_general(q_ref[...].astype(jnp.bfloat16),
                                       kb.astype(jnp.bfloat16),
                                       (((1,), (1,)), ((), ())),
                                       preferred_element_type=jnp.float32)

    colid = jax.lax.broadcasted_iota(jnp.int32, (Q, BK), 1)
    valid = (colid + j * BK) < K
    thresh = rc_ref[:, TOPK - 1:TOPK]
    s = jnp.where(valid & (cos > thresh), cos, NEG)
    s_ref[...] = s
    cnt = jnp.sum((s > NEG).astype(jnp.int32), axis=1, keepdims=True)
    trips = jnp.minimum(jnp.max(cnt), TOPK)

    lanep = jax.lax.broadcasted_iota(jnp.int32, (Q, PADW), 1)

    def ext_body(_, carry):
        sb = s_ref[...]
        m = jnp.max(sb, axis=1, keepdims=True)
        cc = jnp.where(sb == m, colid, jnp.int32(2**30))
        col = jnp.min(cc, axis=1, keepdims=True)
        hit = colid == col
        s_ref[...] = jnp.where(hit, NEG, sb)
        rawsel = jnp.max(jnp.where(hit, raw_ref[...], NEG), axis=1,
                         keepdims=True)
        gidx = col + j * BK
        rv = rc_ref[...]
        ri = ri_ref[...]
        rk = rk_ref[...]
        ge = (rv >= m) & (lanep < TOPK)
        pos = jnp.sum(ge.astype(jnp.int32), axis=1, keepdims=True)
        rv_s = jnp.concatenate([jnp.full((Q, 1), NEG, jnp.float32),
                                rv[:, :PADW - 1]], axis=1)
        ri_s = jnp.concatenate([jnp.zeros((Q, 1), jnp.int32),
                                ri[:, :PADW - 1]], axis=1)
        rk_s = jnp.concatenate([jnp.zeros((Q, 1), jnp.float32),
                                rk[:, :PADW - 1]], axis=1)
        at = lanep == pos
        lt = lanep < pos
        rc_ref[...] = jnp.where(lt, rv, jnp.where(at, m, rv_s))
        ri_ref[...] = jnp.where(lt, ri, jnp.where(at, gidx, ri_s))
        rk_ref[...] = jnp.where(lt, rk, jnp.where(at, rawsel, rk_s))
        return carry

    jax.lax.fori_loop(0, trips, ext_body, 0)

    @pl.when(j == nb - 1)
    def _fin():
        rv = rc_ref[:, :TOPK]
        ri = ri_ref[:, :TOPK]
        rr = rk_ref[:, :TOPK]
        mx = jnp.max(rr, axis=1, keepdims=True)
        lane10 = jax.lax.broadcasted_iota(jnp.int32, (Q, TOPK), 1)
        cc = jnp.where(rr == mx, lane10, jnp.int32(99))
        mi = jnp.min(cc, axis=1, keepdims=True)
        ans = jnp.sum(jnp.where(lane10 == mi, ri, 0), axis=1, keepdims=True)
        ans = jnp.where(mx > 0.5, ans, -1)
        ms_ref[...] = mx
        ti_ref[...] = ri
        rr_ref[...] = rr
        mi_ref[...] = mi
        ai_ref[...] = ans
        ts_ref[...] = rv


def kernel(queries, keys, k):
    Q, D = queries.shape
    K = keys.shape[0]
    nb = pl.cdiv(K, BK)
    out_shape = [
        jax.ShapeDtypeStruct((Q, 1), jnp.float32),
        jax.ShapeDtypeStruct((Q, TOPK), jnp.int32),
        jax.ShapeDtypeStruct((Q, TOPK), jnp.float32),
        jax.ShapeDtypeStruct((Q, 1), jnp.int32),
        jax.ShapeDtypeStruct((Q, 1), jnp.int32),
        jax.ShapeDtypeStruct((Q, TOPK), jnp.float32),
    ]
    outs = pl.pallas_call(
        functools.partial(_retrieval_kernel, K=K),
        grid=(nb,),
        in_specs=[
            pl.BlockSpec((Q, D), lambda j: (0, 0)),
            pl.BlockSpec((BK, D), lambda j: (j, 0)),
        ],
        out_specs=[pl.BlockSpec(s.shape, lambda j: (0, 0)) for s in out_shape],
        out_shape=out_shape,
        scratch_shapes=[
            pltpu.VMEM((Q, BK), jnp.float32),
            pltpu.VMEM((Q, BK), jnp.float32),
            pltpu.VMEM((Q, D), jnp.float32),
            pltpu.VMEM((Q, PADW), jnp.float32),
            pltpu.VMEM((Q, PADW), jnp.int32),
            pltpu.VMEM((Q, PADW), jnp.float32),
        ],
    )(queries, keys)
    ms, ti, rr, mi, ai, ts = outs
    return (ms[:, 0], ti, rr, mi[:, 0], ai[:, 0], ts)
